# Initial kernel scaffold; baseline (speedup 1.0000x reference)
import jax
import jax.numpy as jnp
from jax.experimental import pallas as pl


def kernel(x, edge_index, W1, b1, W2, b2):
    raise NotImplementedError("write your pallas kernel here")


# speedup vs baseline: 14.3746x; 14.3746x over previous
import dataclasses
import functools

import jax
import jax.numpy as jnp
from jax import lax
from jax.experimental import pallas as pl
from jax.experimental.pallas import tpu as pltpu
from jax.experimental.pallas import tpu_sc as plsc

N = 10000
NP = 10240
D = 128
E = 320000
NC = 2
NS = 16
NW = NC * NS
EPW = E // NW
CW = 80
NCHUNK = EPW // CW
RPT = NP // NS
F32 = jnp.float32


def _vmesh():
    return plsc.VectorSubcoreMesh(core_axis_name="c", subcore_axis_name="s")


def _sc_params():
    cp = pltpu.CompilerParams()
    if "needs_layout_passes" in pltpu.CompilerParams.__dataclass_fields__:
        cp = dataclasses.replace(cp, needs_layout_passes=False)
    return cp


@functools.partial(
    pl.kernel,
    out_type=jax.ShapeDtypeStruct((NW, N), F32),
    mesh=_vmesh(),
    compiler_params=_sc_params(),
    scratch_types=[
        pltpu.VMEM((EPW,), jnp.int32),
        pltpu.VMEM((N,), F32),
    ],
)
def _deg_kernel(dst_hbm, out_hbm, dst_v, hist_v):
    cid = lax.axis_index("c")
    sid = lax.axis_index("s")
    wid = cid * NS + sid

    pltpu.sync_copy(dst_hbm.at[pl.ds(wid * EPW, EPW)], dst_v)

    @pl.loop(0, N // 16)
    def _(i):
        hist_v[pl.ds(i * 16, 16)] = jnp.zeros((16,), F32)

    ones16 = jnp.ones((16,), F32)

    @pl.loop(0, EPW // 16)
    def _(i):
        idx16 = dst_v[pl.ds(i * 16, 16)]
        plsc.addupdate_scatter(hist_v, [idx16], ones16)

    pltpu.sync_copy(hist_v, out_hbm.at[wid])


@functools.partial(
    pl.kernel,
    out_type=jax.ShapeDtypeStruct((NC, NP, D), F32),
    mesh=_vmesh(),
    scratch_types=[
        pltpu.VMEM((CW,), jnp.int32),
        pltpu.VMEM((CW,), jnp.int32),
        pltpu.VMEM((CW, D), F32),
        pltpu.VMEM_SHARED((NP, D), F32),
    ],
)
def _agg_kernel(h_hbm, src_hbm, dst_hbm, zeros_hbm, out_hbm,
                srcc_v, dstc_v, rows_v, acc):
    cid = lax.axis_index("c")
    sid = lax.axis_index("s")
    wid = cid * NS + sid

    pltpu.sync_copy(zeros_hbm.at[pl.ds(sid * RPT, RPT)],
                    acc.at[pl.ds(sid * RPT, RPT)])
    plsc.subcore_barrier()

    @pl.loop(0, NCHUNK)
    def _(j):
        base = wid * EPW + j * CW
        pltpu.sync_copy(src_hbm.at[pl.ds(base, CW)], srcc_v)
        pltpu.sync_copy(dst_hbm.at[pl.ds(base, CW)], dstc_v)
        pltpu.sync_copy(h_hbm.at[srcc_v], rows_v)
        pltpu.sync_copy(rows_v, acc.at[dstc_v], add=True)

    plsc.subcore_barrier()
    pltpu.sync_copy(acc.at[pl.ds(sid * RPT, RPT)],
                    out_hbm.at[cid].at[pl.ds(sid * RPT, RPT)])


_BLK = 1024
_GRID = pl.cdiv(N, _BLK)


def _dv(deg_ref):
    return lax.rsqrt(jnp.sum(deg_ref[...], axis=0) + 1.0)[:, None]


def _mm_scale_body(x_ref, w_ref, deg_ref, h_ref):
    h = jnp.dot(x_ref[...], w_ref[...], preferred_element_type=F32)
    h_ref[...] = h * _dv(deg_ref)


def _tc_mm_scale(x, W1t, deg):
    return pl.pallas_call(
        _mm_scale_body,
        grid=(_GRID,),
        in_specs=[
            pl.BlockSpec((_BLK, D), lambda i: (i, 0)),
            pl.BlockSpec((D, D), lambda i: (0, 0)),
            pl.BlockSpec((NW, _BLK), lambda i: (0, i)),
        ],
        out_specs=pl.BlockSpec((_BLK, D), lambda i: (i, 0)),
        out_shape=jax.ShapeDtypeStruct((N, D), F32),
    )(x, W1t, deg)


def _mid_body(acc_ref, h_ref, deg_ref, w_ref, b_ref, o_ref):
    dv = _dv(deg_ref)
    a = acc_ref[...]
    z = dv * (a[0] + a[1] + h_ref[...]) + b_ref[...]
    z = jnp.maximum(z, 0.0)
    o_ref[...] = jnp.dot(z, w_ref[...], preferred_element_type=F32) * dv


def _tc_mid(acc, h1p, deg, W2t, b1r):
    return pl.pallas_call(
        _mid_body,
        grid=(_GRID,),
        in_specs=[
            pl.BlockSpec((NC, _BLK, D), lambda i: (0, i, 0)),
            pl.BlockSpec((_BLK, D), lambda i: (i, 0)),
            pl.BlockSpec((NW, _BLK), lambda i: (0, i)),
            pl.BlockSpec((D, D), lambda i: (0, 0)),
            pl.BlockSpec((1, D), lambda i: (0, 0)),
        ],
        out_specs=pl.BlockSpec((_BLK, D), lambda i: (i, 0)),
        out_shape=jax.ShapeDtypeStruct((N, D), F32),
    )(acc, h1p, deg, W2t, b1r)


def _out_body(acc_ref, h_ref, deg_ref, b_ref, o_ref):
    a = acc_ref[...]
    o_ref[...] = _dv(deg_ref) * (a[0] + a[1] + h_ref[...]) + b_ref[...]


def _tc_out(acc, h2p, deg, b2r):
    return pl.pallas_call(
        _out_body,
        grid=(_GRID,),
        in_specs=[
            pl.BlockSpec((NC, _BLK, D), lambda i: (0, i, 0)),
            pl.BlockSpec((_BLK, D), lambda i: (i, 0)),
            pl.BlockSpec((NW, _BLK), lambda i: (0, i)),
            pl.BlockSpec((1, D), lambda i: (0, 0)),
        ],
        out_specs=pl.BlockSpec((_BLK, D), lambda i: (i, 0)),
        out_shape=jax.ShapeDtypeStruct((N, D), F32),
    )(acc, h2p, deg, b2r)


def kernel(x, edge_index, W1, b1, W2, b2):
    src = edge_index[0].astype(jnp.int32)
    dst = edge_index[1].astype(jnp.int32)
    W1t = W1.T.astype(F32)
    W2t = W2.T.astype(F32)
    b1r = b1.reshape(1, D)
    b2r = b2.reshape(1, D)
    zeros_acc = jnp.zeros((NP, D), F32)

    deg = _deg_kernel(dst)
    h1p = _tc_mm_scale(x, W1t, deg)
    acc1 = _agg_kernel(h1p, src, dst, zeros_acc)
    h2p = _tc_mid(acc1, h1p, deg, W2t, b1r)
    acc2 = _agg_kernel(h2p, src, dst, zeros_acc)
    out = _tc_out(acc2, h2p, deg, b2r)
    return out


# speedup vs baseline: 25.6162x; 1.7821x over previous
import dataclasses
import functools

import jax
import jax.numpy as jnp
from jax import lax
from jax.experimental import pallas as pl
from jax.experimental.pallas import tpu as pltpu
from jax.experimental.pallas import tpu_sc as plsc

N = 10000
NP = 10240
D = 128
E = 320000
NC = 2
NS = 16
NW = NC * NS
EPW = E // NW
CW = 80
NCHUNK = EPW // CW
RPT = NP // NS
F32 = jnp.float32


def _vmesh():
    return plsc.VectorSubcoreMesh(core_axis_name="c", subcore_axis_name="s")


def _sc_params():
    cp = pltpu.CompilerParams()
    if "needs_layout_passes" in pltpu.CompilerParams.__dataclass_fields__:
        cp = dataclasses.replace(cp, needs_layout_passes=False)
    return cp


@functools.partial(
    pl.kernel,
    out_type=jax.ShapeDtypeStruct((NW, N), F32),
    mesh=_vmesh(),
    compiler_params=_sc_params(),
    scratch_types=[
        pltpu.VMEM((EPW,), jnp.int32),
        pltpu.VMEM((N,), F32),
    ],
)
def _deg_kernel(dst_hbm, out_hbm, dst_v, hist_v):
    cid = lax.axis_index("c")
    sid = lax.axis_index("s")
    wid = cid * NS + sid

    pltpu.sync_copy(dst_hbm.at[pl.ds(wid * EPW, EPW)], dst_v)

    @pl.loop(0, N // 16)
    def _(i):
        hist_v[pl.ds(i * 16, 16)] = jnp.zeros((16,), F32)

    ones16 = jnp.ones((16,), F32)

    @pl.loop(0, EPW // 16)
    def _(i):
        idx16 = dst_v[pl.ds(i * 16, 16)]
        plsc.addupdate_scatter(hist_v, [idx16], ones16)

    pltpu.sync_copy(hist_v, out_hbm.at[wid])


@functools.partial(
    pl.kernel,
    out_type=jax.ShapeDtypeStruct((NC, NP, D), F32),
    mesh=_vmesh(),
    scratch_types=[
        pltpu.VMEM((EPW,), jnp.int32),
        pltpu.VMEM((EPW,), jnp.int32),
        pltpu.VMEM((CW, D), F32),
        pltpu.VMEM((CW, D), F32),
        pltpu.VMEM_SHARED((NP, D), F32),
        pltpu.SemaphoreType.DMA,
        pltpu.SemaphoreType.DMA,
        pltpu.SemaphoreType.DMA,
        pltpu.SemaphoreType.DMA,
    ],
)
def _agg_kernel(h_hbm, src_hbm, dst_hbm, zeros_hbm, out_hbm,
                src_v, dst_v, rows_a, rows_b, acc, sga, sgb, sza, szb):
    cid = lax.axis_index("c")
    sid = lax.axis_index("s")
    wid = cid * NS + sid

    pltpu.sync_copy(src_hbm.at[pl.ds(wid * EPW, EPW)], src_v)
    pltpu.sync_copy(dst_hbm.at[pl.ds(wid * EPW, EPW)], dst_v)
    pltpu.sync_copy(zeros_hbm.at[pl.ds(sid * RPT, RPT)],
                    acc.at[pl.ds(sid * RPT, RPT)])
    plsc.subcore_barrier()

    def gs(c, buf, sem):
        pltpu.async_copy(h_hbm.at[src_v.at[pl.ds(c * CW, CW)]], buf, sem)

    def gw(c, buf, sem):
        pltpu.make_async_copy(h_hbm.at[src_v.at[pl.ds(c * CW, CW)]], buf, sem).wait()

    def zs(c, buf, sem):
        pltpu.async_copy(buf, acc.at[dst_v.at[pl.ds(c * CW, CW)]], sem, add=True)

    def zw(c, buf, sem):
        pltpu.make_async_copy(buf, acc.at[dst_v.at[pl.ds(c * CW, CW)]], sem).wait()

    gs(0, rows_a, sga)

    @pl.loop(0, NCHUNK // 2)
    def _(i):
        c0 = 2 * i
        c1 = c0 + 1
        gw(c0, rows_a, sga)

        @pl.when(i > 0)
        def _():
            zw(c1 - 2, rows_b, szb)

        gs(c1, rows_b, sgb)
        zs(c0, rows_a, sza)
        gw(c1, rows_b, sgb)
        zw(c0, rows_a, sza)
        gs(c0 + 2, rows_a, sga)
        zs(c1, rows_b, szb)

    cl = NCHUNK - 1
    gw(cl, rows_a, sga)
    zw(cl - 1, rows_b, szb)
    zs(cl, rows_a, sza)
    zw(cl, rows_a, sza)

    plsc.subcore_barrier()
    pltpu.sync_copy(acc.at[pl.ds(sid * RPT, RPT)],
                    out_hbm.at[cid].at[pl.ds(sid * RPT, RPT)])


_BLK = 1024
_GRID = pl.cdiv(N, _BLK)


def _dv(deg_ref):
    return lax.rsqrt(jnp.sum(deg_ref[...], axis=0) + 1.0)[:, None]


def _mm_scale_body(x_ref, w_ref, deg_ref, h_ref):
    h = jnp.dot(x_ref[...], w_ref[...], preferred_element_type=F32)
    h_ref[...] = h * _dv(deg_ref)


def _tc_mm_scale(x, W1t, deg):
    return pl.pallas_call(
        _mm_scale_body,
        grid=(_GRID,),
        in_specs=[
            pl.BlockSpec((_BLK, D), lambda i: (i, 0)),
            pl.BlockSpec((D, D), lambda i: (0, 0)),
            pl.BlockSpec((NW, _BLK), lambda i: (0, i)),
        ],
        out_specs=pl.BlockSpec((_BLK, D), lambda i: (i, 0)),
        out_shape=jax.ShapeDtypeStruct((N, D), F32),
    )(x, W1t, deg)


def _mid_body(acc_ref, h_ref, deg_ref, w_ref, b_ref, o_ref):
    dv = _dv(deg_ref)
    a = acc_ref[...]
    z = dv * (a[0] + a[1] + h_ref[...]) + b_ref[...]
    z = jnp.maximum(z, 0.0)
    o_ref[...] = jnp.dot(z, w_ref[...], preferred_element_type=F32) * dv


def _tc_mid(acc, h1p, deg, W2t, b1r):
    return pl.pallas_call(
        _mid_body,
        grid=(_GRID,),
        in_specs=[
            pl.BlockSpec((NC, _BLK, D), lambda i: (0, i, 0)),
            pl.BlockSpec((_BLK, D), lambda i: (i, 0)),
            pl.BlockSpec((NW, _BLK), lambda i: (0, i)),
            pl.BlockSpec((D, D), lambda i: (0, 0)),
            pl.BlockSpec((1, D), lambda i: (0, 0)),
        ],
        out_specs=pl.BlockSpec((_BLK, D), lambda i: (i, 0)),
        out_shape=jax.ShapeDtypeStruct((N, D), F32),
    )(acc, h1p, deg, W2t, b1r)


def _out_body(acc_ref, h_ref, deg_ref, b_ref, o_ref):
    a = acc_ref[...]
    o_ref[...] = _dv(deg_ref) * (a[0] + a[1] + h_ref[...]) + b_ref[...]


def _tc_out(acc, h2p, deg, b2r):
    return pl.pallas_call(
        _out_body,
        grid=(_GRID,),
        in_specs=[
            pl.BlockSpec((NC, _BLK, D), lambda i: (0, i, 0)),
            pl.BlockSpec((_BLK, D), lambda i: (i, 0)),
            pl.BlockSpec((NW, _BLK), lambda i: (0, i)),
            pl.BlockSpec((1, D), lambda i: (0, 0)),
        ],
        out_specs=pl.BlockSpec((_BLK, D), lambda i: (i, 0)),
        out_shape=jax.ShapeDtypeStruct((N, D), F32),
    )(acc, h2p, deg, b2r)


def kernel(x, edge_index, W1, b1, W2, b2):
    src = edge_index[0].astype(jnp.int32)
    dst = edge_index[1].astype(jnp.int32)
    W1t = W1.T.astype(F32)
    W2t = W2.T.astype(F32)
    b1r = b1.reshape(1, D)
    b2r = b2.reshape(1, D)
    zeros_acc = jnp.zeros((NP, D), F32)

    deg = _deg_kernel(dst)
    h1p = _tc_mm_scale(x, W1t, deg)
    acc1 = _agg_kernel(h1p, src, dst, zeros_acc)
    h2p = _tc_mid(acc1, h1p, deg, W2t, b1r)
    acc2 = _agg_kernel(h2p, src, dst, zeros_acc)
    out = _tc_out(acc2, h2p, deg, b2r)
    return out


# speedup vs baseline: 27.1689x; 1.0606x over previous
import dataclasses
import functools

import jax
import jax.numpy as jnp
from jax import lax
from jax.experimental import pallas as pl
from jax.experimental.pallas import tpu as pltpu
from jax.experimental.pallas import tpu_sc as plsc

N = 10000
NP = 10240
D = 128
E = 320000
NC = 2
NS = 16
NW = NC * NS
EPW = E // NW
CW = 96
NCHUNK = EPW // CW
TAIL = EPW - NCHUNK * CW
RPT = NP // NS
F32 = jnp.float32


def _vmesh():
    return plsc.VectorSubcoreMesh(core_axis_name="c", subcore_axis_name="s")


def _sc_params():
    cp = pltpu.CompilerParams()
    if "needs_layout_passes" in pltpu.CompilerParams.__dataclass_fields__:
        cp = dataclasses.replace(cp, needs_layout_passes=False)
    return cp


@functools.partial(
    pl.kernel,
    out_type=jax.ShapeDtypeStruct((NW, N), F32),
    mesh=_vmesh(),
    compiler_params=_sc_params(),
    scratch_types=[
        pltpu.VMEM((EPW,), jnp.int32),
        pltpu.VMEM((N,), F32),
    ],
)
def _deg_kernel(dst_hbm, out_hbm, dst_v, hist_v):
    cid = lax.axis_index("c")
    sid = lax.axis_index("s")
    wid = cid * NS + sid

    pltpu.sync_copy(dst_hbm.at[pl.ds(wid * EPW, EPW)], dst_v)

    @pl.loop(0, N // 16)
    def _(i):
        hist_v[pl.ds(i * 16, 16)] = jnp.zeros((16,), F32)

    ones16 = jnp.ones((16,), F32)

    @pl.loop(0, EPW // 16)
    def _(i):
        idx16 = dst_v[pl.ds(i * 16, 16)]
        plsc.addupdate_scatter(hist_v, [idx16], ones16)

    pltpu.sync_copy(hist_v, out_hbm.at[wid])


@functools.partial(
    pl.kernel,
    out_type=jax.ShapeDtypeStruct((NC, NP, D), F32),
    mesh=_vmesh(),
    scratch_types=[
        pltpu.VMEM((EPW,), jnp.int32),
        pltpu.VMEM((EPW,), jnp.int32),
        pltpu.VMEM((CW, D), F32),
        pltpu.VMEM((CW, D), F32),
        pltpu.VMEM_SHARED((NP, D), F32),
        pltpu.SemaphoreType.DMA,
        pltpu.SemaphoreType.DMA,
        pltpu.SemaphoreType.DMA,
        pltpu.SemaphoreType.DMA,
    ],
)
def _agg_kernel(h_hbm, src_hbm, dst_hbm, zeros_hbm, out_hbm,
                src_v, dst_v, rows_a, rows_b, acc, sga, sgb, sza, szb):
    cid = lax.axis_index("c")
    sid = lax.axis_index("s")
    wid = cid * NS + sid

    pltpu.sync_copy(src_hbm.at[pl.ds(wid * EPW, EPW)], src_v)
    pltpu.sync_copy(dst_hbm.at[pl.ds(wid * EPW, EPW)], dst_v)
    pltpu.sync_copy(zeros_hbm.at[pl.ds(sid * RPT, RPT)],
                    acc.at[pl.ds(sid * RPT, RPT)])
    plsc.subcore_barrier()

    def gs(c, buf, sem):
        pltpu.async_copy(h_hbm.at[src_v.at[pl.ds(c * CW, CW)]], buf, sem)

    def gw(c, buf, sem):
        pltpu.make_async_copy(h_hbm.at[src_v.at[pl.ds(c * CW, CW)]], buf, sem).wait()

    def zs(c, buf, sem):
        pltpu.async_copy(buf, acc.at[dst_v.at[pl.ds(c * CW, CW)]], sem, add=True)

    def zw(c, buf, sem):
        pltpu.make_async_copy(buf, acc.at[dst_v.at[pl.ds(c * CW, CW)]], sem).wait()

    gs(0, rows_a, sga)

    @pl.loop(0, NCHUNK // 2)
    def _(i):
        c0 = 2 * i
        c1 = c0 + 1
        gw(c0, rows_a, sga)

        @pl.when(i > 0)
        def _():
            zw(c1 - 2, rows_b, szb)

        gs(c1, rows_b, sgb)
        zs(c0, rows_a, sza)
        gw(c1, rows_b, sgb)
        zw(c0, rows_a, sza)
        gs(jnp.where(c0 + 2 < NCHUNK, c0 + 2, 0), rows_a, sga)
        zs(c1, rows_b, szb)

    gw(0, rows_a, sga)
    zw(NCHUNK - 1, rows_b, szb)

    tb = NCHUNK * CW
    pltpu.async_copy(h_hbm.at[src_v.at[pl.ds(tb, TAIL)]],
                     rows_a.at[pl.ds(0, TAIL)], sga)
    pltpu.make_async_copy(h_hbm.at[src_v.at[pl.ds(tb, TAIL)]],
                          rows_a.at[pl.ds(0, TAIL)], sga).wait()
    pltpu.sync_copy(rows_a.at[pl.ds(0, TAIL)],
                    acc.at[dst_v.at[pl.ds(tb, TAIL)]], add=True)

    plsc.subcore_barrier()
    pltpu.sync_copy(acc.at[pl.ds(sid * RPT, RPT)],
                    out_hbm.at[cid].at[pl.ds(sid * RPT, RPT)])


_BLK = 1024
_GRID = pl.cdiv(N, _BLK)


def _dv(deg_ref):
    return lax.rsqrt(jnp.sum(deg_ref[...], axis=0) + 1.0)[:, None]


def _mm_scale_body(x_ref, w_ref, deg_ref, h_ref):
    h = jnp.dot(x_ref[...], w_ref[...], preferred_element_type=F32)
    h_ref[...] = h * _dv(deg_ref)


def _tc_mm_scale(x, W1t, deg):
    return pl.pallas_call(
        _mm_scale_body,
        grid=(_GRID,),
        in_specs=[
            pl.BlockSpec((_BLK, D), lambda i: (i, 0)),
            pl.BlockSpec((D, D), lambda i: (0, 0)),
            pl.BlockSpec((NW, _BLK), lambda i: (0, i)),
        ],
        out_specs=pl.BlockSpec((_BLK, D), lambda i: (i, 0)),
        out_shape=jax.ShapeDtypeStruct((N, D), F32),
    )(x, W1t, deg)


def _mid_body(acc_ref, h_ref, deg_ref, w_ref, b_ref, o_ref):
    dv = _dv(deg_ref)
    a = acc_ref[...]
    z = dv * (a[0] + a[1] + h_ref[...]) + b_ref[...]
    z = jnp.maximum(z, 0.0)
    o_ref[...] = jnp.dot(z, w_ref[...], preferred_element_type=F32) * dv


def _tc_mid(acc, h1p, deg, W2t, b1r):
    return pl.pallas_call(
        _mid_body,
        grid=(_GRID,),
        in_specs=[
            pl.BlockSpec((NC, _BLK, D), lambda i: (0, i, 0)),
            pl.BlockSpec((_BLK, D), lambda i: (i, 0)),
            pl.BlockSpec((NW, _BLK), lambda i: (0, i)),
            pl.BlockSpec((D, D), lambda i: (0, 0)),
            pl.BlockSpec((1, D), lambda i: (0, 0)),
        ],
        out_specs=pl.BlockSpec((_BLK, D), lambda i: (i, 0)),
        out_shape=jax.ShapeDtypeStruct((N, D), F32),
    )(acc, h1p, deg, W2t, b1r)


def _out_body(acc_ref, h_ref, deg_ref, b_ref, o_ref):
    a = acc_ref[...]
    o_ref[...] = _dv(deg_ref) * (a[0] + a[1] + h_ref[...]) + b_ref[...]


def _tc_out(acc, h2p, deg, b2r):
    return pl.pallas_call(
        _out_body,
        grid=(_GRID,),
        in_specs=[
            pl.BlockSpec((NC, _BLK, D), lambda i: (0, i, 0)),
            pl.BlockSpec((_BLK, D), lambda i: (i, 0)),
            pl.BlockSpec((NW, _BLK), lambda i: (0, i)),
            pl.BlockSpec((1, D), lambda i: (0, 0)),
        ],
        out_specs=pl.BlockSpec((_BLK, D), lambda i: (i, 0)),
        out_shape=jax.ShapeDtypeStruct((N, D), F32),
    )(acc, h2p, deg, b2r)


def kernel(x, edge_index, W1, b1, W2, b2):
    src = edge_index[0].astype(jnp.int32)
    dst = edge_index[1].astype(jnp.int32)
    W1t = W1.T.astype(F32)
    W2t = W2.T.astype(F32)
    b1r = b1.reshape(1, D)
    b2r = b2.reshape(1, D)
    zeros_acc = jnp.zeros((NP, D), F32)

    deg = _deg_kernel(dst)
    h1p = _tc_mm_scale(x, W1t, deg)
    acc1 = _agg_kernel(h1p, src, dst, zeros_acc)
    h2p = _tc_mid(acc1, h1p, deg, W2t, b1r)
    acc2 = _agg_kernel(h2p, src, dst, zeros_acc)
    out = _tc_out(acc2, h2p, deg, b2r)
    return out
